# baseline (speedup 1.0000x reference)
import jax
import jax.numpy as jnp
from jax import lax
from jax.experimental import pallas as pl
from jax.experimental.pallas import tpu as pltpu
from jax.experimental.pallas import tpu_sc as plsc

B, H, W_IN, C_CH = 8, 112, 112, 96
M = B * H * W_IN * C_CH
N = M * 4
NC, NS = 2, 16
NP = 12
WWIN = N // (NP * NC)
SEG = M // NS
CHUNK = 5376
NCHUNK = SEG // CHUNK
ROWS = CHUNK // 128
RW = 128
RC = 16
LN = 256
MAXOUT = 8
FLUSH = WWIN // NS
ZCH = 2048
NZ = FLUSH // ZCH


def _body(idx_hbm, upd_hbm, out_hbm, win_ref, i0, v0, i1, v1, cidx, cval,
          line_i, line_v, zbuf, sem0, sem1, semz, sems):
    cid = lax.axis_index("c")
    sid = lax.axis_index("s")
    zeros16 = jnp.zeros((16,), jnp.float32)
    iota16 = lax.iota(jnp.int32, 16)

    def zero_zbuf(i, _):
        zbuf[pl.ds(i * 16, 16)] = zeros16
        return 0

    lax.fori_loop(0, ZCH // 16, zero_zbuf, 0)

    def fire(ch, ib, vb, sem):
        off = sid * SEG + ch * CHUNK
        pltpu.async_copy(idx_hbm.at[pl.ds(off, CHUNK)], ib, sem)
        pltpu.async_copy(upd_hbm.at[pl.ds(off, CHUNK)], vb, sem)

    def wait(ib, vb, sem):
        pltpu.make_async_copy(idx_hbm.at[pl.ds(0, CHUNK)], ib, sem).wait()
        pltpu.make_async_copy(upd_hbm.at[pl.ds(0, CHUNK)], vb, sem).wait()

    def wait_scat():
        pltpu.make_async_copy(cval.at[0], win_ref.at[cidx.at[0]],
                              sems).wait()

    def one_pass(p, _):
        win = p * NC + cid
        base = win * WWIN

        fire(0, i0, v0, sem0)
        fire(1, i1, v1, sem1)

        def zfill(z, _):
            pltpu.async_copy(
                zbuf, win_ref.at[pl.ds(sid * FLUSH + z * ZCH, ZCH)], semz)
            return 0

        lax.fori_loop(0, NZ, zfill, 0)

        def zwait(z, _):
            pltpu.make_async_copy(
                zbuf, win_ref.at[pl.ds(sid * FLUSH, ZCH)], semz).wait()
            return 0

        lax.fori_loop(0, NZ, zwait, 0)
        plsc.subcore_barrier()

        def process(ib, vb, q, row, waited):
            def one_row(r, carry):
                q, row, waited = carry
                tv, uvv, hitv, pcs = [], [], [], []
                for c in range(8):
                    sl = pl.ds(r * 128 + c * 16, 16)
                    t = ib[sl] - base
                    uv = vb[sl]
                    hit = plsc.bitcast(t, jnp.uint32) < jnp.uint32(WWIN)
                    pc = plsc.all_reduce_population_count(hit)
                    tv.append(t)
                    uvv.append(uv)
                    hitv.append(hit)
                    pcs.append(pc[0])
                offs = [q]
                for c in range(8):
                    offs.append(offs[c] + pcs[c])
                for c in range(8):
                    plsc.store_compressed(line_i.at[pl.ds(offs[c], 16)],
                                          tv[c], mask=hitv[c])
                    plsc.store_compressed(line_v.at[pl.ds(offs[c], 16)],
                                          uvv[c], mask=hitv[c])
                qn = offs[8]

                @pl.when(qn >= RW)
                def _():
                    rr = jnp.bitwise_and(row, RC - 1)
                    for k in range(8):
                        sl = pl.ds(k * 16, 16)
                        cidx[rr, sl] = line_i[sl]
                        cval[rr, sl] = line_v[sl]
                    for k in range(8):
                        sl = pl.ds(k * 16, 16)
                        shl = pl.ds(RW + k * 16, 16)
                        line_i[sl] = line_i[shl]
                        line_v[sl] = line_v[shl]
                    pltpu.async_copy(cval.at[rr], win_ref.at[cidx.at[rr]],
                                     sems, add=True)

                    @pl.when(row - waited >= MAXOUT)
                    def _():
                        wait_scat()

                ovf = qn >= RW
                winc = jnp.logical_and(ovf, row - waited >= MAXOUT)
                q = jnp.where(ovf, qn - RW, qn)
                row = jnp.where(ovf, row + 1, row)
                waited = jnp.where(winc, waited + 1, waited)
                return q, row, waited

            return lax.fori_loop(0, ROWS, one_row, (q, row, waited))

        def two_chunks(g, carry):
            q, row, waited = carry
            wait(i0, v0, sem0)
            q, row, waited = process(i0, v0, q, row, waited)

            @pl.when(g < NCHUNK // 2 - 1)
            def _():
                fire(2 * g + 2, i0, v0, sem0)

            wait(i1, v1, sem1)
            q, row, waited = process(i1, v1, q, row, waited)

            @pl.when(g < NCHUNK // 2 - 1)
            def _():
                fire(2 * g + 3, i1, v1, sem1)

            return q, row, waited

        q, row, waited = lax.fori_loop(
            0, NCHUNK // 2, two_chunks,
            (jnp.int32(0), jnp.int32(0), jnp.int32(0)))

        def dwait(s, _):
            wait_scat()
            return 0

        lax.fori_loop(waited, row, dwait, 0)

        @pl.when(q > 0)
        def _():
            rr = jnp.bitwise_and(row, RC - 1)
            for j in range(RW // 16):
                slc = pl.ds(j * 16, 16)
                keep = (iota16 + j * 16) < q
                cval[rr, slc] = jnp.where(keep, line_v[slc], 0.0)
                cidx[rr, slc] = jnp.where(keep, line_i[slc],
                                          (iota16 + j * 16) * 52)
            pltpu.sync_copy(cval.at[rr], win_ref.at[cidx.at[rr]], add=True)

        plsc.subcore_barrier()

        pltpu.sync_copy(
            win_ref.at[pl.ds(sid * FLUSH, FLUSH)],
            out_hbm.at[pl.ds(base + sid * FLUSH, FLUSH)],
        )
        plsc.subcore_barrier()
        return 0

    lax.fori_loop(0, NP, one_pass, 0)


@jax.jit
def kernel(updates, mask):
    flat_idx = jnp.reshape(mask, (-1,)).astype(jnp.int32)
    flat_upd = jnp.reshape(updates, (-1,))
    mesh = plsc.VectorSubcoreMesh(core_axis_name="c", subcore_axis_name="s")
    out = pl.kernel(
        _body,
        compiler_params=pltpu.CompilerParams(needs_layout_passes=False),
        out_type=jax.ShapeDtypeStruct((N,), jnp.float32),
        mesh=mesh,
        scratch_types=[
            pltpu.VMEM_SHARED((WWIN,), jnp.float32),
            pltpu.VMEM((CHUNK,), jnp.int32),
            pltpu.VMEM((CHUNK,), jnp.float32),
            pltpu.VMEM((CHUNK,), jnp.int32),
            pltpu.VMEM((CHUNK,), jnp.float32),
            pltpu.VMEM((RC, RW), jnp.int32),
            pltpu.VMEM((RC, RW), jnp.float32),
            pltpu.VMEM((LN,), jnp.int32),
            pltpu.VMEM((LN,), jnp.float32),
            pltpu.VMEM((ZCH,), jnp.float32),
            pltpu.SemaphoreType.DMA,
            pltpu.SemaphoreType.DMA,
            pltpu.SemaphoreType.DMA,
            pltpu.SemaphoreType.DMA,
        ],
    )(flat_idx, flat_upd)
    return jnp.reshape(out, (B, H * 2, W_IN * 2, C_CH))
